# Initial kernel scaffold; baseline (speedup 1.0000x reference)
import jax
import jax.numpy as jnp
from jax.experimental import pallas as pl


def kernel(fc_log, genotypes, expression_obs, variantxgene_to_gene, local_variant_to_local_variantxgene_selector, variantxgene_to_local_gene, lib, baseline_log, dispersion_log):
    raise NotImplementedError("write your pallas kernel here")


# speedup vs baseline: 8.2197x; 8.2197x over previous
import functools

import jax
import jax.numpy as jnp
from jax import lax
from jax.experimental import pallas as pl
from jax.experimental.pallas import tpu as pltpu
from jax.experimental.pallas import tpu_sc as plsc

_NC = 2
_NS = 16
_NW = _NC * _NS
_CH = 128


def _sc_gather(baseline_t, genotypes_t, vxg2g, sel):
    v_total = vxg2g.shape[0]
    per_w = v_total // _NW
    n_ch = per_w // _CH

    mesh = plsc.VectorSubcoreMesh(core_axis_name="c", subcore_axis_name="s")

    @functools.partial(
        pl.kernel,
        out_type=(
            jax.ShapeDtypeStruct((v_total, baseline_t.shape[1]), jnp.float32),
            jax.ShapeDtypeStruct((v_total, genotypes_t.shape[1]), jnp.float32),
        ),
        mesh=mesh,
        scratch_types=[
            pltpu.VMEM((_CH,), jnp.int32),
            pltpu.VMEM((_CH,), jnp.int32),
            pltpu.VMEM((_CH, baseline_t.shape[1]), jnp.float32),
            pltpu.VMEM((_CH, genotypes_t.shape[1]), jnp.float32),
            pltpu.SemaphoreType.DMA,
            pltpu.SemaphoreType.DMA,
        ],
        compiler_params=pltpu.CompilerParams(use_tc_tiling_on_sc=False),
    )
    def gather_kernel(bt_hbm, gt_hbm, vg_hbm, sel_hbm, outb_hbm, outg_hbm,
                      idx_b, idx_g, rows_b, rows_g, sem_b, sem_g):
        wid = lax.axis_index("s") * _NC + lax.axis_index("c")
        base = wid * per_w
        for j in range(n_ch):
            off = base + j * _CH
            pltpu.sync_copy(vg_hbm.at[pl.ds(off, _CH)], idx_b)
            pltpu.sync_copy(sel_hbm.at[pl.ds(off, _CH)], idx_g)
            cp_b = pltpu.async_copy(bt_hbm.at[idx_b], rows_b, sem_b)
            cp_g = pltpu.async_copy(gt_hbm.at[idx_g], rows_g, sem_g)
            cp_b.wait()
            cp_g.wait()
            pltpu.sync_copy(rows_b, outb_hbm.at[pl.ds(off, _CH)])
            pltpu.sync_copy(rows_g, outg_hbm.at[pl.ds(off, _CH)])

    return gather_kernel(baseline_t, genotypes_t, vxg2g, sel)


def _tc_dense(fc_log, bsel, gsel, lib3):
    n_c, v_total = fc_log.shape
    n_d = gsel.shape[0]
    v_blk = 512
    grid = (v_total // v_blk,)

    def body(fc_ref, b_ref, g_ref, lib_ref, out_ref):
        fc = fc_ref[...]
        b = b_ref[:n_c, :]
        g = g_ref[...]
        t = b[None, :, :] + g[:, None, :] * fc[None, :, :]
        out_ref[...] = jnp.exp(t) * lib_ref[...]

    return pl.pallas_call(
        body,
        grid=grid,
        in_specs=[
            pl.BlockSpec((n_c, v_blk), lambda i: (0, i)),
            pl.BlockSpec((bsel.shape[0], v_blk), lambda i: (0, i)),
            pl.BlockSpec((n_d, v_blk), lambda i: (0, i)),
            pl.BlockSpec((n_d, n_c, 1), lambda i: (0, 0, 0)),
        ],
        out_specs=pl.BlockSpec((n_d, n_c, v_blk), lambda i: (0, 0, i)),
        out_shape=jax.ShapeDtypeStruct((n_d, n_c, v_total), jnp.float32),
    )(fc_log, bsel, gsel, lib3)


def kernel(fc_log, genotypes, expression_obs, variantxgene_to_gene,
           local_variant_to_local_variantxgene_selector, variantxgene_to_local_gene,
           lib, baseline_log, dispersion_log):
    del expression_obs, variantxgene_to_local_gene, dispersion_log
    n_c = fc_log.shape[0]
    n_d = genotypes.shape[0]

    baseline_t = jnp.pad(baseline_log.T, ((0, 0), (0, 32 - n_c)))
    genotypes_t = genotypes.T

    bsel_t, gsel_t = _sc_gather(
        baseline_t, genotypes_t,
        variantxgene_to_gene, local_variant_to_local_variantxgene_selector)

    bsel = bsel_t.T
    gsel = gsel_t.T
    return _tc_dense(fc_log, bsel, gsel, lib.reshape(n_d, n_c, 1))
